# baseline (speedup 1.0000x reference)
import functools

import jax
import jax.numpy as jnp
from jax import lax
from jax.experimental import pallas as pl
from jax.experimental.pallas import tpu as pltpu
from jax.experimental.pallas import tpu_sc as plsc

N_NODES = 10000
N_EDGES = 320000
D_EDGE = 16
EMB = 128

NC = 2
NS = 16
NW = NC * NS
EPW = N_EDGES // NW
CH = 80
NCHUNK = EPW // CH
NPAD = 10240
RPT = NPAD // NS

_EB = 4000
_RB = 1000


def _edge_embed(edge_attr, Wcat, bcat):
  def body(a_ref, w_ref, b_ref, o1_ref, o2_ref):
    e = jnp.dot(a_ref[...], w_ref[...],
                preferred_element_type=jnp.float32) + b_ref[...]
    o1_ref[...] = e[:, :EMB]
    o2_ref[...] = e[:, EMB:]

  return pl.pallas_call(
      body,
      grid=(N_EDGES // _EB,),
      in_specs=[
          pl.BlockSpec((_EB, D_EDGE), lambda i: (i, 0)),
          pl.BlockSpec((D_EDGE, 2 * EMB), lambda i: (0, 0)),
          pl.BlockSpec((1, 2 * EMB), lambda i: (0, 0)),
      ],
      out_specs=[
          pl.BlockSpec((_EB, EMB), lambda i: (i, 0)),
          pl.BlockSpec((_EB, EMB), lambda i: (i, 0)),
      ],
      out_shape=[jax.ShapeDtypeStruct((N_EDGES, EMB), jnp.float32)] * 2,
  )(edge_attr, Wcat, bcat)


_SC_MESH = plsc.VectorSubcoreMesh(core_axis_name="c", subcore_axis_name="s")


@functools.partial(
    pl.kernel,
    out_type=jax.ShapeDtypeStruct((NC * NPAD, EMB), jnp.float32),
    mesh=_SC_MESH,
    scratch_types=[
        pltpu.VMEM((2, CH), jnp.int32),
        pltpu.VMEM((2, CH), jnp.int32),
        pltpu.VMEM((2, CH), jnp.int32),
        pltpu.VMEM((2, CH, EMB), jnp.float32),
        pltpu.VMEM((2, CH, EMB), jnp.float32),
        pltpu.VMEM_SHARED((NPAD, EMB), jnp.float32),
        [pltpu.SemaphoreType.DMA] * 2,
        [pltpu.SemaphoreType.DMA] * 2,
        [pltpu.SemaphoreType.DMA] * 2,
        [pltpu.SemaphoreType.DMA] * 2,
    ],
)
def _sc_segment(h_hbm, e_hbm, src_hbm, dst_hbm, z_hbm, out_hbm,
                sidx, didx, sdst, hrows, erows, aggsh, gsem, esem, isem, ssem):
  c = lax.axis_index("c")
  s = lax.axis_index("s")
  wid = c * NS + s
  ebase = wid * EPW

  zcp = pltpu.async_copy(z_hbm, aggsh.at[pl.ds(s * RPT, RPT)], gsem[0])

  pltpu.sync_copy(src_hbm.at[pl.ds(ebase, CH)], sidx.at[0])
  pltpu.sync_copy(dst_hbm.at[pl.ds(ebase, CH)], didx.at[0])
  pltpu.async_copy(src_hbm.at[pl.ds(ebase + CH, CH)], sidx.at[1], isem[1])
  pltpu.async_copy(dst_hbm.at[pl.ds(ebase + CH, CH)], didx.at[1], isem[1])
  zcp.wait()
  pltpu.async_copy(h_hbm.at[sidx.at[0]], hrows.at[0], gsem[0])
  pltpu.async_copy(e_hbm.at[pl.ds(ebase, CH)], erows.at[0], esem[0])
  plsc.subcore_barrier()

  def _when(cond):
    def deco(fn):
      if isinstance(cond, (bool, int)):
        if cond:
          fn()
        return fn
      return pl.when(cond)(fn)
    return deco

  def _do_chunk(t, b):
    b2 = 1 - b

    pltpu.make_async_copy(h_hbm.at[pl.ds(0, CH)], hrows.at[b], gsem[b]).wait()
    pltpu.make_async_copy(e_hbm.at[pl.ds(0, CH)], erows.at[b], esem[b]).wait()

    @pl.loop(0, CH, unroll=4)
    def _row(r):
      for j in range(EMB // 16):
        sl = pl.ds(j * 16, 16)
        hrows[b, r, sl] = jnp.maximum(hrows[b, r, sl] + erows[b, r, sl], 0.0)

    @_when(t >= 1)
    def _():
      pltpu.make_async_copy(h_hbm.at[pl.ds(0, CH)], hrows.at[b2],
                            ssem[b2]).wait()

    @_when(t + 1 < NCHUNK)
    def _():
      pltpu.make_async_copy(src_hbm.at[pl.ds(0, CH)], sidx.at[b2],
                            isem[b2]).wait()
      pltpu.make_async_copy(dst_hbm.at[pl.ds(0, CH)], didx.at[b2],
                            isem[b2]).wait()
      pltpu.async_copy(h_hbm.at[sidx.at[b2]], hrows.at[b2], gsem[b2])
      pltpu.async_copy(e_hbm.at[pl.ds(ebase + (t + 1) * CH, CH)],
                       erows.at[b2], esem[b2])

    for k in range(CH // 16):
      sl = pl.ds(k * 16, 16)
      sdst[b, sl] = didx[b, sl]
    pltpu.async_copy(hrows.at[b], aggsh.at[sdst.at[b]], ssem[b], add=True)

    @_when(t + 2 < NCHUNK)
    def _():
      nbase = ebase + (t + 2) * CH
      pltpu.async_copy(src_hbm.at[pl.ds(nbase, CH)], sidx.at[b], isem[b])
      pltpu.async_copy(dst_hbm.at[pl.ds(nbase, CH)], didx.at[b], isem[b])

  @pl.loop(0, NCHUNK // 2)
  def _pair(i):
    t0 = 2 * i
    _do_chunk(t0, 0)
    _do_chunk(t0 + 1, 1)

  if NCHUNK % 2:
    _do_chunk(NCHUNK - 1, 0)

  pltpu.make_async_copy(h_hbm.at[pl.ds(0, CH)],
                        hrows.at[(NCHUNK - 1) % 2],
                        ssem[(NCHUNK - 1) % 2]).wait()

  plsc.subcore_barrier()
  pltpu.sync_copy(aggsh.at[pl.ds(s * RPT, RPT)],
                  out_hbm.at[pl.ds(c * NPAD + s * RPT, RPT)])


def _mlp(h, parts, alpha, W1f, b1f, W2f, b2f, relu_out):
  def body(al_ref, h_ref, p_ref, w1_ref, b1_ref, w2_ref, b2_ref, o_ref):
    t = h_ref[...] * al_ref[0, 0] + p_ref[0] + p_ref[1]
    t = jnp.dot(t, w1_ref[...], preferred_element_type=jnp.float32) + b1_ref[...]
    t = jnp.maximum(t, 0.0)
    t = jnp.dot(t, w2_ref[...], preferred_element_type=jnp.float32) + b2_ref[...]
    if relu_out:
      t = jnp.maximum(t, 0.0)
    o_ref[...] = t

  return pl.pallas_call(
      body,
      grid=(N_NODES // _RB,),
      in_specs=[
          pl.BlockSpec((1, 1), lambda i: (0, 0)),
          pl.BlockSpec((_RB, EMB), lambda i: (i, 0)),
          pl.BlockSpec((NC, _RB, EMB), lambda i: (0, i, 0)),
          pl.BlockSpec((EMB, 2 * EMB), lambda i: (0, 0)),
          pl.BlockSpec((1, 2 * EMB), lambda i: (0, 0)),
          pl.BlockSpec((2 * EMB, EMB), lambda i: (0, 0)),
          pl.BlockSpec((1, EMB), lambda i: (0, 0)),
      ],
      out_specs=pl.BlockSpec((_RB, EMB), lambda i: (i, 0)),
      out_shape=jax.ShapeDtypeStruct((N_NODES, EMB), jnp.float32),
  )(alpha, h, parts, W1f, b1f[None], W2f, b2f[None])


def kernel(x, edge_index, edge_attr, We, be, eps, W1, b1, W2, b2,
           g1, bb1, m1, v1, go, bo, mo, vo):
  s1 = g1 / jnp.sqrt(v1 + 1e-5)
  W1f = W1 * s1[:, None, :]
  b1f = (b1 - m1) * s1 + bb1
  so = go / jnp.sqrt(vo + 1e-5)
  W2f = W2 * so[:, None, :]
  b2f = (b2 - mo) * so + bo

  Wcat = jnp.concatenate([We[0], We[1]], axis=1)
  bcat = jnp.concatenate([be[0], be[1]])[None, :]
  E1, E2 = _edge_embed(edge_attr, Wcat, bcat)

  src = edge_index[0]
  dst = edge_index[1]
  z = jnp.zeros((RPT, EMB), jnp.float32)

  h = x
  for l in range(2):
    El = E1 if l == 0 else E2
    parts = _sc_segment(h, El, src, dst, z).reshape(NC, NPAD, EMB)
    alpha = (1.0 + eps[l]).reshape(1, 1)
    h = _mlp(h, parts, alpha, W1f[l], b1f[l], W2f[l], b2f[l],
             relu_out=(l == 0))
  return h


# speedup vs baseline: 1.0023x; 1.0023x over previous
import functools

import jax
import jax.numpy as jnp
import numpy as np
from jax import lax
from jax.experimental import pallas as pl
from jax.experimental.pallas import tpu as pltpu
from jax.experimental.pallas import tpu_sc as plsc

N_NODES = 10000
N_EDGES = 320000
D_EDGE = 16
EMB = 128

NC = 2
NS = 16
NW = NC * NS
EPW = N_EDGES // NW
CH = 80
NCHUNK = EPW // CH
NPAD = 10240
RPT = NPAD // NS

_EB = 4000
_RB = 1000

def _edge_embed(edge_attr, Wcat, bcat):
  def body(a_ref, w_ref, b_ref, o1_ref, o2_ref):
    e = jnp.dot(a_ref[...], w_ref[...],
                preferred_element_type=jnp.float32) + b_ref[...]
    o1_ref[...] = e[:, :EMB]
    o2_ref[...] = e[:, EMB:]

  return pl.pallas_call(
      body,
      grid=(N_EDGES // _EB,),
      in_specs=[
          pl.BlockSpec((_EB, D_EDGE), lambda i: (i, 0)),
          pl.BlockSpec((D_EDGE, 2 * EMB), lambda i: (0, 0)),
          pl.BlockSpec((1, 2 * EMB), lambda i: (0, 0)),
      ],
      out_specs=[
          pl.BlockSpec((_EB, EMB), lambda i: (i, 0)),
          pl.BlockSpec((_EB, EMB), lambda i: (i, 0)),
      ],
      out_shape=[jax.ShapeDtypeStruct((N_EDGES, EMB), jnp.float32)] * 2,
  )(edge_attr, Wcat, bcat)


_SC_MESH = plsc.VectorSubcoreMesh(core_axis_name="c", subcore_axis_name="s")


@functools.partial(
    pl.kernel,
    out_type=jax.ShapeDtypeStruct((NC * NPAD, EMB), jnp.float32),
    mesh=_SC_MESH,
    scratch_types=[
        pltpu.VMEM((2, CH), jnp.int32),
        pltpu.VMEM((2, CH), jnp.int32),
        pltpu.VMEM((2, CH), jnp.int32),
        pltpu.VMEM((2, CH, EMB), jnp.float32),
        pltpu.VMEM((2, CH, EMB), jnp.float32),
        pltpu.VMEM_SHARED((NPAD, EMB), jnp.float32),
        [pltpu.SemaphoreType.DMA] * 2,
        [pltpu.SemaphoreType.DMA] * 2,
        [pltpu.SemaphoreType.DMA] * 2,
        [pltpu.SemaphoreType.DMA] * 2,
    ],
)
def _sc_segment(h_hbm, e_hbm, src_hbm, dst_hbm, z_hbm, out_hbm,
                sidx, didx, sdst, hrows, erows, aggsh,
                gsem, esem, isem, ssem):
  c = lax.axis_index("c")
  s = lax.axis_index("s")
  wid = c * NS + s
  ebase = wid * EPW

  zcp = pltpu.async_copy(z_hbm, aggsh.at[pl.ds(s * RPT, RPT)], gsem[0])

  pltpu.sync_copy(src_hbm.at[pl.ds(ebase, CH)], sidx.at[0])
  pltpu.sync_copy(dst_hbm.at[pl.ds(ebase, CH)], didx.at[0])
  pltpu.async_copy(src_hbm.at[pl.ds(ebase + CH, CH)], sidx.at[1], isem[1])
  pltpu.async_copy(dst_hbm.at[pl.ds(ebase + CH, CH)], didx.at[1], isem[1])
  zcp.wait()
  pltpu.async_copy(h_hbm.at[sidx.at[0]], hrows.at[0], gsem[0])
  pltpu.async_copy(e_hbm.at[pl.ds(ebase, CH)], erows.at[0], esem[0])
  plsc.subcore_barrier()

  def _when(cond):
    def deco(fn):
      if isinstance(cond, (bool, int)):
        if cond:
          fn()
        return fn
      return pl.when(cond)(fn)
    return deco

  def _do_chunk(t, b):
    b2 = 1 - b

    pltpu.make_async_copy(h_hbm.at[pl.ds(0, CH)], hrows.at[b], gsem[b]).wait()
    pltpu.make_async_copy(e_hbm.at[pl.ds(0, CH)], erows.at[b], esem[b]).wait()

    @pl.loop(0, CH, unroll=4)
    def _row(r):
      for g in range(EMB // 16):
        sl = pl.ds(g * 16, 16)
        hrows[b, r, sl] = jnp.maximum(hrows[b, r, sl] + erows[b, r, sl], 0.0)

    @_when(t >= 1)
    def _():
      pltpu.make_async_copy(out_hbm.at[pl.ds(0, CH)], hrows.at[b2],
                            ssem[b2]).wait()

    @_when(t + 1 < NCHUNK)
    def _():
      pltpu.make_async_copy(src_hbm.at[pl.ds(0, CH)], sidx.at[b2],
                            isem[b2]).wait()
      pltpu.make_async_copy(dst_hbm.at[pl.ds(0, CH)], didx.at[b2],
                            isem[b2]).wait()
      pltpu.async_copy(h_hbm.at[sidx.at[b2]], hrows.at[b2], gsem[b2])
      pltpu.async_copy(e_hbm.at[pl.ds(ebase + (t + 1) * CH, CH)],
                       erows.at[b2], esem[b2])

    for k in range(CH // 16):
      sl = pl.ds(k * 16, 16)
      sdst[b, sl] = didx[b, sl]
    pltpu.async_copy(hrows.at[b], aggsh.at[sdst.at[b]], ssem[b], add=True)

    @_when(t + 2 < NCHUNK)
    def _():
      nbase = ebase + (t + 2) * CH
      pltpu.async_copy(src_hbm.at[pl.ds(nbase, CH)], sidx.at[b], isem[b])
      pltpu.async_copy(dst_hbm.at[pl.ds(nbase, CH)], didx.at[b], isem[b])

  @pl.loop(0, NCHUNK // 2)
  def _pair(i):
    t0 = 2 * i
    _do_chunk(t0, 0)
    _do_chunk(t0 + 1, 1)

  if NCHUNK % 2:
    _do_chunk(NCHUNK - 1, 0)

  pltpu.make_async_copy(out_hbm.at[pl.ds(0, CH)],
                        hrows.at[(NCHUNK - 1) % 2],
                        ssem[(NCHUNK - 1) % 2]).wait()

  plsc.subcore_barrier()
  pltpu.sync_copy(aggsh.at[pl.ds(s * RPT, RPT)],
                  out_hbm.at[pl.ds(c * NPAD + s * RPT, RPT)])


def _mlp(h, parts, alpha, W1f, b1f, W2f, b2f, relu_out, Wtw=None, btw=None):
  twin = Wtw is not None

  def body(al_ref, h_ref, p_ref, w1_ref, b1_ref, w2_ref, b2_ref,
           *rest):
    t = h_ref[...] * al_ref[0, 0] + p_ref[0] + p_ref[1]
    u = jnp.dot(t, w1_ref[...], preferred_element_type=jnp.float32) + b1_ref[...]
    u = jnp.maximum(u, 0.0)
    t = jnp.dot(u, w2_ref[...], preferred_element_type=jnp.float32) + b2_ref[...]
    if relu_out:
      t = jnp.maximum(t, 0.0)
    if twin:
      wt_ref, bt_ref, o_ref, o2_ref = rest
      tp = jnp.dot(u, wt_ref[...], preferred_element_type=jnp.float32) + bt_ref[...]
      if relu_out:
        tp = jnp.maximum(tp, 0.0)
      o2_ref[...] = tp.astype(jnp.bfloat16)
    else:
      (o_ref,) = rest
    o_ref[...] = t

  in_specs = [
      pl.BlockSpec((1, 1), lambda i: (0, 0)),
      pl.BlockSpec((_RB, EMB), lambda i: (i, 0)),
      pl.BlockSpec((NC, _RB, EMB), lambda i: (0, i, 0)),
      pl.BlockSpec((EMB, 2 * EMB), lambda i: (0, 0)),
      pl.BlockSpec((1, 2 * EMB), lambda i: (0, 0)),
      pl.BlockSpec((2 * EMB, EMB), lambda i: (0, 0)),
      pl.BlockSpec((1, EMB), lambda i: (0, 0)),
  ]
  args = [alpha, h, parts, W1f, b1f[None], W2f, b2f[None]]
  out_specs = pl.BlockSpec((_RB, EMB), lambda i: (i, 0))
  out_shape = jax.ShapeDtypeStruct((N_NODES, EMB), jnp.float32)
  if twin:
    in_specs += [pl.BlockSpec((2 * EMB, EMB), lambda i: (0, 0)),
                 pl.BlockSpec((1, EMB), lambda i: (0, 0))]
    args += [Wtw, btw[None]]
    out_specs = [out_specs, pl.BlockSpec((_RB, EMB), lambda i: (i, 0))]
    out_shape = [out_shape,
                 jax.ShapeDtypeStruct((N_NODES, EMB), jnp.bfloat16)]

  return pl.pallas_call(
      body,
      grid=(N_NODES // _RB,),
      in_specs=in_specs,
      out_specs=out_specs,
      out_shape=out_shape,
  )(*args)


def kernel(x, edge_index, edge_attr, We, be, eps, W1, b1, W2, b2,
           g1, bb1, m1, v1, go, bo, mo, vo):
  s1 = g1 / jnp.sqrt(v1 + 1e-5)
  W1f = W1 * s1[:, None, :]
  b1f = (b1 - m1) * s1 + bb1
  so = go / jnp.sqrt(vo + 1e-5)
  W2f = W2 * so[:, None, :]
  b2f = (b2 - mo) * so + bo

  Wcat = jnp.concatenate([We[0], We[1]], axis=1)
  bcat = jnp.concatenate([be[0], be[1]])[None, :]
  E1, E2 = _edge_embed(edge_attr, Wcat, bcat)

  src = edge_index[0]
  dst = edge_index[1]
  z = jnp.zeros((RPT, EMB), jnp.float32)

  parts = _sc_segment(x, E1, src, dst, z).reshape(NC, NPAD, EMB)
  h = _mlp(x, parts, (1.0 + eps[0]).reshape(1, 1),
           W1f[0], b1f[0], W2f[0], b2f[0], relu_out=True)
  parts = _sc_segment(h, E2, src, dst, z).reshape(NC, NPAD, EMB)
  h = _mlp(h, parts, (1.0 + eps[1]).reshape(1, 1),
           W1f[1], b1f[1], W2f[1], b2f[1], relu_out=False)
  return h


# speedup vs baseline: 1.8838x; 1.8795x over previous
import functools

import jax
import jax.numpy as jnp
import numpy as np
from jax import lax
from jax.experimental import pallas as pl
from jax.experimental.pallas import tpu as pltpu
from jax.experimental.pallas import tpu_sc as plsc

N_NODES = 10000
N_EDGES = 320000
D_EDGE = 16
EMB = 128

NC = 2
NS = 16
NW = NC * NS
EPW = N_EDGES // NW
CH = 80
NCHUNK = EPW // CH
NPAD = 10240
RPT = NPAD // NS

_EB = 4000
_RB = 1000

def _edge_embed(edge_attr, Wcat, bcat):
  def body(a_ref, w_ref, b_ref, o1_ref, o2_ref):
    e = jnp.dot(a_ref[...], w_ref[...],
                preferred_element_type=jnp.float32) + b_ref[...]
    o1_ref[...] = e[:, :EMB]
    o2_ref[...] = e[:, EMB:]

  return pl.pallas_call(
      body,
      grid=(N_EDGES // _EB,),
      in_specs=[
          pl.BlockSpec((_EB, D_EDGE), lambda i: (i, 0)),
          pl.BlockSpec((D_EDGE, 2 * EMB), lambda i: (0, 0)),
          pl.BlockSpec((1, 2 * EMB), lambda i: (0, 0)),
      ],
      out_specs=[
          pl.BlockSpec((_EB, EMB), lambda i: (i, 0)),
          pl.BlockSpec((_EB, EMB), lambda i: (i, 0)),
      ],
      out_shape=[jax.ShapeDtypeStruct((N_EDGES, EMB), jnp.float32)] * 2,
  )(edge_attr, Wcat, bcat)


_SC_MESH = plsc.VectorSubcoreMesh(core_axis_name="c", subcore_axis_name="s")


@functools.partial(
    pl.kernel,
    out_type=jax.ShapeDtypeStruct((NC * NPAD, EMB), jnp.float32),
    mesh=_SC_MESH,
    scratch_types=[
        pltpu.VMEM((2, CH), jnp.int32),
        pltpu.VMEM((2, CH), jnp.int32),
        pltpu.VMEM((2, CH), jnp.int32),
        pltpu.VMEM((3, CH, EMB), jnp.float32),
        pltpu.VMEM_SHARED((NPAD, EMB), jnp.float32),
        [pltpu.SemaphoreType.DMA] * 3,
        [pltpu.SemaphoreType.DMA] * 3,
        [pltpu.SemaphoreType.DMA] * 2,
        [pltpu.SemaphoreType.DMA] * 2,
        [pltpu.SemaphoreType.DMA] * 1,
    ],
)
def _sc_segment(h_hbm, e_hbm, src_hbm, dst_hbm, z_hbm, out_hbm,
                sidx, didx, sdst, rows, aggsh,
                gsem, esem, isem, ssem, zsem):
  c = lax.axis_index("c")
  s = lax.axis_index("s")
  wid = c * NS + s
  ebase = wid * EPW

  zcp = pltpu.async_copy(z_hbm, aggsh.at[pl.ds(s * RPT, RPT)], zsem[0])

  pltpu.sync_copy(src_hbm.at[pl.ds(ebase, CH)], sidx.at[0])
  pltpu.sync_copy(dst_hbm.at[pl.ds(ebase, CH)], didx.at[0])
  pltpu.async_copy(src_hbm.at[pl.ds(ebase + CH, CH)], sidx.at[1], isem[1])
  pltpu.async_copy(dst_hbm.at[pl.ds(ebase + CH, CH)], didx.at[1], isem[1])
  for k in range(3):
    pltpu.async_copy(e_hbm.at[pl.ds(ebase + k * CH, CH)], rows.at[k], esem[k])
  zcp.wait()
  pltpu.make_async_copy(e_hbm.at[pl.ds(0, CH)], rows.at[0], esem[0]).wait()
  pltpu.async_copy(h_hbm.at[sidx.at[0]], rows.at[0], gsem[0], add=True)
  plsc.subcore_barrier()

  def _do_chunk(t, r3, r2, first, e_ok, g_ok, i_ok):
    r3p = (r3 + 2) % 3
    r3n = (r3 + 1) % 3
    r2n = 1 - r2

    pltpu.make_async_copy(h_hbm.at[pl.ds(0, CH)], rows.at[r3], gsem[r3]).wait()

    @pl.loop(0, CH, unroll=4)
    def _row(r):
      for g in range(EMB // 16):
        sl = pl.ds(g * 16, 16)
        rows[r3, r, sl] = jnp.maximum(rows[r3, r, sl], 0.0)

    if not first:
      pltpu.make_async_copy(out_hbm.at[pl.ds(0, CH)], rows.at[r3p],
                            ssem[r2n]).wait()
    if e_ok:
      pltpu.async_copy(e_hbm.at[pl.ds(ebase + (t + 2) * CH, CH)],
                       rows.at[r3p], esem[r3p])

    for k in range(CH // 16):
      sl = pl.ds(k * 16, 16)
      sdst[r2, sl] = didx[r2, sl]
    pltpu.async_copy(rows.at[r3], aggsh.at[sdst.at[r2]], ssem[r2], add=True)

    if g_ok:
      pltpu.make_async_copy(e_hbm.at[pl.ds(0, CH)], rows.at[r3n],
                            esem[r3n]).wait()
      pltpu.make_async_copy(src_hbm.at[pl.ds(0, CH)], sidx.at[r2n],
                            isem[r2n]).wait()
      pltpu.make_async_copy(dst_hbm.at[pl.ds(0, CH)], didx.at[r2n],
                            isem[r2n]).wait()
      pltpu.async_copy(h_hbm.at[sidx.at[r2n]], rows.at[r3n], gsem[r3n],
                       add=True)

    if i_ok:
      nbase = ebase + (t + 2) * CH
      pltpu.async_copy(src_hbm.at[pl.ds(nbase, CH)], sidx.at[r2], isem[r2])
      pltpu.async_copy(dst_hbm.at[pl.ds(nbase, CH)], didx.at[r2], isem[r2])

  for t in range(6):
    _do_chunk(t, t % 3, t % 2, t == 0, t >= 1, True, True)

  @pl.loop(1, NCHUNK // 6)
  def _grp(i):
    t0 = 6 * i
    for j in range(6):
      _do_chunk(t0 + j, j % 3, j % 2, False, True, True, True)

  for t in range(6 * (NCHUNK // 6), NCHUNK):
    _do_chunk(t, t % 3, t % 2, False, t + 2 < NCHUNK, t + 1 < NCHUNK,
              t + 2 < NCHUNK)

  pltpu.make_async_copy(out_hbm.at[pl.ds(0, CH)],
                        rows.at[(NCHUNK - 1) % 3],
                        ssem[(NCHUNK - 1) % 2]).wait()

  plsc.subcore_barrier()
  pltpu.sync_copy(aggsh.at[pl.ds(s * RPT, RPT)],
                  out_hbm.at[pl.ds(c * NPAD + s * RPT, RPT)])


def _mlp(h, parts, alpha, W1f, b1f, W2f, b2f, relu_out, Wtw=None, btw=None):
  twin = Wtw is not None

  def body(al_ref, h_ref, p_ref, w1_ref, b1_ref, w2_ref, b2_ref,
           *rest):
    t = h_ref[...] * al_ref[0, 0] + p_ref[0] + p_ref[1]
    u = jnp.dot(t, w1_ref[...], preferred_element_type=jnp.float32) + b1_ref[...]
    u = jnp.maximum(u, 0.0)
    t = jnp.dot(u, w2_ref[...], preferred_element_type=jnp.float32) + b2_ref[...]
    if relu_out:
      t = jnp.maximum(t, 0.0)
    if twin:
      wt_ref, bt_ref, o_ref, o2_ref = rest
      tp = jnp.dot(u, wt_ref[...], preferred_element_type=jnp.float32) + bt_ref[...]
      if relu_out:
        tp = jnp.maximum(tp, 0.0)
      o2_ref[...] = tp.astype(jnp.bfloat16)
    else:
      (o_ref,) = rest
    o_ref[...] = t

  in_specs = [
      pl.BlockSpec((1, 1), lambda i: (0, 0)),
      pl.BlockSpec((_RB, EMB), lambda i: (i, 0)),
      pl.BlockSpec((NC, _RB, EMB), lambda i: (0, i, 0)),
      pl.BlockSpec((EMB, 2 * EMB), lambda i: (0, 0)),
      pl.BlockSpec((1, 2 * EMB), lambda i: (0, 0)),
      pl.BlockSpec((2 * EMB, EMB), lambda i: (0, 0)),
      pl.BlockSpec((1, EMB), lambda i: (0, 0)),
  ]
  args = [alpha, h, parts, W1f, b1f[None], W2f, b2f[None]]
  out_specs = pl.BlockSpec((_RB, EMB), lambda i: (i, 0))
  out_shape = jax.ShapeDtypeStruct((N_NODES, EMB), jnp.float32)
  if twin:
    in_specs += [pl.BlockSpec((2 * EMB, EMB), lambda i: (0, 0)),
                 pl.BlockSpec((1, EMB), lambda i: (0, 0))]
    args += [Wtw, btw[None]]
    out_specs = [out_specs, pl.BlockSpec((_RB, EMB), lambda i: (i, 0))]
    out_shape = [out_shape,
                 jax.ShapeDtypeStruct((N_NODES, EMB), jnp.bfloat16)]

  return pl.pallas_call(
      body,
      grid=(N_NODES // _RB,),
      in_specs=in_specs,
      out_specs=out_specs,
      out_shape=out_shape,
  )(*args)


def kernel(x, edge_index, edge_attr, We, be, eps, W1, b1, W2, b2,
           g1, bb1, m1, v1, go, bo, mo, vo):
  s1 = g1 / jnp.sqrt(v1 + 1e-5)
  W1f = W1 * s1[:, None, :]
  b1f = (b1 - m1) * s1 + bb1
  so = go / jnp.sqrt(vo + 1e-5)
  W2f = W2 * so[:, None, :]
  b2f = (b2 - mo) * so + bo

  Wcat = jnp.concatenate([We[0], We[1]], axis=1)
  bcat = jnp.concatenate([be[0], be[1]])[None, :]
  E1, E2 = _edge_embed(edge_attr, Wcat, bcat)

  src = edge_index[0]
  dst = edge_index[1]
  z = jnp.zeros((RPT, EMB), jnp.float32)

  parts = _sc_segment(x, E1, src, dst, z).reshape(NC, NPAD, EMB)
  h = _mlp(x, parts, (1.0 + eps[0]).reshape(1, 1),
           W1f[0], b1f[0], W2f[0], b2f[0], relu_out=True)
  parts = _sc_segment(h, E2, src, dst, z).reshape(NC, NPAD, EMB)
  h = _mlp(h, parts, (1.0 + eps[1]).reshape(1, 1),
           W1f[1], b1f[1], W2f[1], b2f[1], relu_out=False)
  return h


# speedup vs baseline: 1.8957x; 1.0064x over previous
import functools

import jax
import jax.numpy as jnp
import numpy as np
from jax import lax
from jax.experimental import pallas as pl
from jax.experimental.pallas import tpu as pltpu
from jax.experimental.pallas import tpu_sc as plsc

N_NODES = 10000
N_EDGES = 320000
D_EDGE = 16
EMB = 128

NC = 2
NS = 16
NW = NC * NS
EPW = N_EDGES // NW
CH = 80
NCHUNK = EPW // CH
NPAD = 10240
RPT = NPAD // NS

_EB = 4000
_RB = 1000

def _edge_embed(edge_attr, Wcat, bcat):
  def body(a_ref, w_ref, b_ref, o1_ref, o2_ref):
    e = jnp.dot(a_ref[...], w_ref[...],
                preferred_element_type=jnp.float32) + b_ref[...]
    o1_ref[...] = e[:, :EMB]
    o2_ref[...] = e[:, EMB:]

  return pl.pallas_call(
      body,
      grid=(N_EDGES // _EB,),
      in_specs=[
          pl.BlockSpec((_EB, D_EDGE), lambda i: (i, 0)),
          pl.BlockSpec((D_EDGE, 2 * EMB), lambda i: (0, 0)),
          pl.BlockSpec((1, 2 * EMB), lambda i: (0, 0)),
      ],
      out_specs=[
          pl.BlockSpec((_EB, EMB), lambda i: (i, 0)),
          pl.BlockSpec((_EB, EMB), lambda i: (i, 0)),
      ],
      out_shape=[jax.ShapeDtypeStruct((N_EDGES, EMB), jnp.float32)] * 2,
  )(edge_attr, Wcat, bcat)


_SC_MESH = plsc.VectorSubcoreMesh(core_axis_name="c", subcore_axis_name="s")


@functools.partial(
    pl.kernel,
    out_type=jax.ShapeDtypeStruct((NC * NPAD, EMB), jnp.float32),
    mesh=_SC_MESH,
    scratch_types=[
        pltpu.VMEM((2, CH), jnp.int32),
        pltpu.VMEM((2, CH), jnp.int32),
        pltpu.VMEM((2, CH), jnp.int32),
        pltpu.VMEM((3, CH, EMB), jnp.float32),
        pltpu.VMEM_SHARED((NPAD, EMB), jnp.float32),
        [pltpu.SemaphoreType.DMA] * 3,
        [pltpu.SemaphoreType.DMA] * 3,
        [pltpu.SemaphoreType.DMA] * 2,
        [pltpu.SemaphoreType.DMA] * 2,
        [pltpu.SemaphoreType.DMA] * 1,
    ],
)
def _sc_segment(h_hbm, e_hbm, src_hbm, dst_hbm, z_hbm, out_hbm,
                sidx, didx, sdst, rows, aggsh,
                gsem, esem, isem, ssem, zsem):
  c = lax.axis_index("c")
  s = lax.axis_index("s")
  wid = c * NS + s
  ebase = wid * EPW

  zcp = pltpu.async_copy(z_hbm, aggsh.at[pl.ds(s * RPT, RPT)], zsem[0])

  pltpu.sync_copy(src_hbm.at[pl.ds(ebase, CH)], sidx.at[0])
  pltpu.sync_copy(dst_hbm.at[pl.ds(ebase, CH)], didx.at[0])
  pltpu.async_copy(src_hbm.at[pl.ds(ebase + CH, CH)], sidx.at[1], isem[1])
  pltpu.async_copy(dst_hbm.at[pl.ds(ebase + CH, CH)], didx.at[1], isem[1])
  for k in range(3):
    pltpu.async_copy(e_hbm.at[pl.ds(ebase + k * CH, CH)], rows.at[k], esem[k])
  zcp.wait()
  pltpu.make_async_copy(e_hbm.at[pl.ds(0, CH)], rows.at[0], esem[0]).wait()
  pltpu.async_copy(h_hbm.at[sidx.at[0]], rows.at[0], gsem[0], add=True)
  plsc.subcore_barrier()

  def _do_chunk(t, r3, r2, first, e_ok, g_ok, i_ok):
    r3p = (r3 + 2) % 3
    r3n = (r3 + 1) % 3
    r2n = 1 - r2

    pltpu.make_async_copy(h_hbm.at[pl.ds(0, CH)], rows.at[r3], gsem[r3]).wait()

    @pl.loop(0, CH, unroll=4)
    def _row(r):
      rows[r3, r] = jnp.maximum(rows[r3, r], 0.0)

    if not first:
      pltpu.make_async_copy(out_hbm.at[pl.ds(0, CH)], rows.at[r3p],
                            ssem[r2n]).wait()
    if e_ok:
      pltpu.async_copy(e_hbm.at[pl.ds(ebase + (t + 2) * CH, CH)],
                       rows.at[r3p], esem[r3p])

    for k in range(CH // 16):
      sl = pl.ds(k * 16, 16)
      sdst[r2, sl] = didx[r2, sl]
    pltpu.async_copy(rows.at[r3], aggsh.at[sdst.at[r2]], ssem[r2], add=True)

    if g_ok:
      pltpu.make_async_copy(e_hbm.at[pl.ds(0, CH)], rows.at[r3n],
                            esem[r3n]).wait()
      pltpu.make_async_copy(src_hbm.at[pl.ds(0, CH)], sidx.at[r2n],
                            isem[r2n]).wait()
      pltpu.make_async_copy(dst_hbm.at[pl.ds(0, CH)], didx.at[r2n],
                            isem[r2n]).wait()
      pltpu.async_copy(h_hbm.at[sidx.at[r2n]], rows.at[r3n], gsem[r3n],
                       add=True)

    if i_ok:
      nbase = ebase + (t + 2) * CH
      pltpu.async_copy(src_hbm.at[pl.ds(nbase, CH)], sidx.at[r2], isem[r2])
      pltpu.async_copy(dst_hbm.at[pl.ds(nbase, CH)], didx.at[r2], isem[r2])

  for t in range(6):
    _do_chunk(t, t % 3, t % 2, t == 0, t >= 1, True, True)

  @pl.loop(1, NCHUNK // 6)
  def _grp(i):
    t0 = 6 * i
    for j in range(6):
      _do_chunk(t0 + j, j % 3, j % 2, False, True, True, True)

  for t in range(6 * (NCHUNK // 6), NCHUNK):
    _do_chunk(t, t % 3, t % 2, False, t + 2 < NCHUNK, t + 1 < NCHUNK,
              t + 2 < NCHUNK)

  pltpu.make_async_copy(out_hbm.at[pl.ds(0, CH)],
                        rows.at[(NCHUNK - 1) % 3],
                        ssem[(NCHUNK - 1) % 2]).wait()

  plsc.subcore_barrier()
  pltpu.sync_copy(aggsh.at[pl.ds(s * RPT, RPT)],
                  out_hbm.at[pl.ds(c * NPAD + s * RPT, RPT)])


def _mlp(h, parts, alpha, W1f, b1f, W2f, b2f, relu_out, Wtw=None, btw=None):
  twin = Wtw is not None

  def body(al_ref, h_ref, p_ref, w1_ref, b1_ref, w2_ref, b2_ref,
           *rest):
    t = h_ref[...] * al_ref[0, 0] + p_ref[0] + p_ref[1]
    u = jnp.dot(t, w1_ref[...], preferred_element_type=jnp.float32) + b1_ref[...]
    u = jnp.maximum(u, 0.0)
    t = jnp.dot(u, w2_ref[...], preferred_element_type=jnp.float32) + b2_ref[...]
    if relu_out:
      t = jnp.maximum(t, 0.0)
    if twin:
      wt_ref, bt_ref, o_ref, o2_ref = rest
      tp = jnp.dot(u, wt_ref[...], preferred_element_type=jnp.float32) + bt_ref[...]
      if relu_out:
        tp = jnp.maximum(tp, 0.0)
      o2_ref[...] = tp.astype(jnp.bfloat16)
    else:
      (o_ref,) = rest
    o_ref[...] = t

  in_specs = [
      pl.BlockSpec((1, 1), lambda i: (0, 0)),
      pl.BlockSpec((_RB, EMB), lambda i: (i, 0)),
      pl.BlockSpec((NC, _RB, EMB), lambda i: (0, i, 0)),
      pl.BlockSpec((EMB, 2 * EMB), lambda i: (0, 0)),
      pl.BlockSpec((1, 2 * EMB), lambda i: (0, 0)),
      pl.BlockSpec((2 * EMB, EMB), lambda i: (0, 0)),
      pl.BlockSpec((1, EMB), lambda i: (0, 0)),
  ]
  args = [alpha, h, parts, W1f, b1f[None], W2f, b2f[None]]
  out_specs = pl.BlockSpec((_RB, EMB), lambda i: (i, 0))
  out_shape = jax.ShapeDtypeStruct((N_NODES, EMB), jnp.float32)
  if twin:
    in_specs += [pl.BlockSpec((2 * EMB, EMB), lambda i: (0, 0)),
                 pl.BlockSpec((1, EMB), lambda i: (0, 0))]
    args += [Wtw, btw[None]]
    out_specs = [out_specs, pl.BlockSpec((_RB, EMB), lambda i: (i, 0))]
    out_shape = [out_shape,
                 jax.ShapeDtypeStruct((N_NODES, EMB), jnp.bfloat16)]

  return pl.pallas_call(
      body,
      grid=(N_NODES // _RB,),
      in_specs=in_specs,
      out_specs=out_specs,
      out_shape=out_shape,
  )(*args)


def kernel(x, edge_index, edge_attr, We, be, eps, W1, b1, W2, b2,
           g1, bb1, m1, v1, go, bo, mo, vo):
  s1 = g1 / jnp.sqrt(v1 + 1e-5)
  W1f = W1 * s1[:, None, :]
  b1f = (b1 - m1) * s1 + bb1
  so = go / jnp.sqrt(vo + 1e-5)
  W2f = W2 * so[:, None, :]
  b2f = (b2 - mo) * so + bo

  Wcat = jnp.concatenate([We[0], We[1]], axis=1)
  bcat = jnp.concatenate([be[0], be[1]])[None, :]
  E1, E2 = _edge_embed(edge_attr, Wcat, bcat)

  src = edge_index[0]
  dst = edge_index[1]
  z = jnp.zeros((RPT, EMB), jnp.float32)

  parts = _sc_segment(x, E1, src, dst, z).reshape(NC, NPAD, EMB)
  h = _mlp(x, parts, (1.0 + eps[0]).reshape(1, 1),
           W1f[0], b1f[0], W2f[0], b2f[0], relu_out=True)
  parts = _sc_segment(h, E2, src, dst, z).reshape(NC, NPAD, EMB)
  h = _mlp(h, parts, (1.0 + eps[1]).reshape(1, 1),
           W1f[1], b1f[1], W2f[1], b2f[1], relu_out=False)
  return h


# speedup vs baseline: 2.0394x; 1.0758x over previous
import functools

import jax
import jax.numpy as jnp
import numpy as np
from jax import lax
from jax.experimental import pallas as pl
from jax.experimental.pallas import tpu as pltpu
from jax.experimental.pallas import tpu_sc as plsc

N_NODES = 10000
N_EDGES = 320000
D_EDGE = 16
EMB = 128

NC = 2
NS = 16
NW = NC * NS
EPW = N_EDGES // NW
CH = 80
NCHUNK = EPW // CH
NPAD = 10240
RPT = NPAD // NS

_EB = 4000
_RB = 1000

def _edge_embed(edge_attr, Wcat, bcat):
  def body(a_ref, w_ref, b_ref, o1_ref, o2_ref):
    e = jnp.dot(a_ref[...], w_ref[...],
                preferred_element_type=jnp.float32) + b_ref[...]
    o1_ref[...] = e[:, :EMB]
    o2_ref[...] = e[:, EMB:]

  return pl.pallas_call(
      body,
      grid=(N_EDGES // _EB,),
      in_specs=[
          pl.BlockSpec((_EB, D_EDGE), lambda i: (i, 0)),
          pl.BlockSpec((D_EDGE, 2 * EMB), lambda i: (0, 0)),
          pl.BlockSpec((1, 2 * EMB), lambda i: (0, 0)),
      ],
      out_specs=[
          pl.BlockSpec((_EB, EMB), lambda i: (i, 0)),
          pl.BlockSpec((_EB, EMB), lambda i: (i, 0)),
      ],
      out_shape=[jax.ShapeDtypeStruct((N_EDGES, EMB), jnp.float32)] * 2,
  )(edge_attr, Wcat, bcat)


_SC_MESH = plsc.VectorSubcoreMesh(core_axis_name="c", subcore_axis_name="s")


@functools.partial(
    pl.kernel,
    out_type=jax.ShapeDtypeStruct((NC * NPAD, EMB), jnp.float32),
    mesh=_SC_MESH,
    scratch_types=[
        pltpu.VMEM((2, CH), jnp.int32),
        pltpu.VMEM((2, CH), jnp.int32),
        pltpu.VMEM((2, CH), jnp.int32),
        pltpu.VMEM((3, CH, EMB), jnp.float32),
        pltpu.VMEM_SHARED((NPAD, EMB), jnp.float32),
        [pltpu.SemaphoreType.DMA] * 3,
        [pltpu.SemaphoreType.DMA] * 3,
        [pltpu.SemaphoreType.DMA] * 2,
        [pltpu.SemaphoreType.DMA] * 2,
        [pltpu.SemaphoreType.DMA] * 1,
    ],
)
def _sc_segment(h_hbm, e_hbm, src_hbm, dst_hbm, z_hbm, out_hbm,
                sidx, didx, sdst, rows, aggsh,
                gsem, esem, isem, ssem, zsem):
  c = lax.axis_index("c")
  s = lax.axis_index("s")
  wid = c * NS + s
  ebase = wid * EPW

  zcp = pltpu.async_copy(z_hbm, aggsh.at[pl.ds(s * RPT, RPT)], zsem[0])

  pltpu.sync_copy(src_hbm.at[pl.ds(ebase, CH)], sidx.at[0])
  pltpu.sync_copy(dst_hbm.at[pl.ds(ebase, CH)], didx.at[0])
  pltpu.async_copy(src_hbm.at[pl.ds(ebase + CH, CH)], sidx.at[1], isem[1])
  pltpu.async_copy(dst_hbm.at[pl.ds(ebase + CH, CH)], didx.at[1], isem[1])
  for k in range(3):
    pltpu.async_copy(e_hbm.at[pl.ds(ebase + k * CH, CH)], rows.at[k], esem[k])
  zcp.wait()
  pltpu.make_async_copy(e_hbm.at[pl.ds(0, CH)], rows.at[0], esem[0]).wait()
  pltpu.async_copy(h_hbm.at[sidx.at[0]], rows.at[0], gsem[0], add=True)
  plsc.subcore_barrier()

  def _do_chunk(t, r3, r2, first, e_ok, g_ok, i_ok):
    r3p = (r3 + 2) % 3
    r3n = (r3 + 1) % 3
    r2n = 1 - r2

    pltpu.make_async_copy(h_hbm.at[pl.ds(0, CH)], rows.at[r3], gsem[r3]).wait()

    if g_ok:
      pltpu.make_async_copy(e_hbm.at[pl.ds(0, CH)], rows.at[r3n],
                            esem[r3n]).wait()
      pltpu.make_async_copy(src_hbm.at[pl.ds(0, CH)], sidx.at[r2n],
                            isem[r2n]).wait()
      pltpu.make_async_copy(dst_hbm.at[pl.ds(0, CH)], didx.at[r2n],
                            isem[r2n]).wait()
      pltpu.async_copy(h_hbm.at[sidx.at[r2n]], rows.at[r3n], gsem[r3n],
                       add=True)

    if not first:
      pltpu.make_async_copy(out_hbm.at[pl.ds(0, CH)], rows.at[r3p],
                            ssem[r2n]).wait()
    if e_ok:
      pltpu.async_copy(e_hbm.at[pl.ds(ebase + (t + 2) * CH, CH)],
                       rows.at[r3p], esem[r3p])

    @pl.loop(0, CH, unroll=4)
    def _row(r):
      rows[r3, r] = jnp.maximum(rows[r3, r], 0.0)

    for k in range(CH // 16):
      sl = pl.ds(k * 16, 16)
      sdst[r2, sl] = didx[r2, sl]
    pltpu.async_copy(rows.at[r3], aggsh.at[sdst.at[r2]], ssem[r2], add=True)

    if i_ok:
      nbase = ebase + (t + 2) * CH
      pltpu.async_copy(src_hbm.at[pl.ds(nbase, CH)], sidx.at[r2], isem[r2])
      pltpu.async_copy(dst_hbm.at[pl.ds(nbase, CH)], didx.at[r2], isem[r2])

  for t in range(6):
    _do_chunk(t, t % 3, t % 2, t == 0, t >= 1, True, True)

  @pl.loop(1, NCHUNK // 6)
  def _grp(i):
    t0 = 6 * i
    for j in range(6):
      _do_chunk(t0 + j, j % 3, j % 2, False, True, True, True)

  for t in range(6 * (NCHUNK // 6), NCHUNK):
    _do_chunk(t, t % 3, t % 2, False, t + 2 < NCHUNK, t + 1 < NCHUNK,
              t + 2 < NCHUNK)

  pltpu.make_async_copy(out_hbm.at[pl.ds(0, CH)],
                        rows.at[(NCHUNK - 1) % 3],
                        ssem[(NCHUNK - 1) % 2]).wait()

  plsc.subcore_barrier()
  pltpu.sync_copy(aggsh.at[pl.ds(s * RPT, RPT)],
                  out_hbm.at[pl.ds(c * NPAD + s * RPT, RPT)])


def _mlp(h, parts, alpha, W1f, b1f, W2f, b2f, relu_out, Wtw=None, btw=None):
  twin = Wtw is not None

  def body(al_ref, h_ref, p_ref, w1_ref, b1_ref, w2_ref, b2_ref,
           *rest):
    t = h_ref[...] * al_ref[0, 0] + p_ref[0] + p_ref[1]
    u = jnp.dot(t, w1_ref[...], preferred_element_type=jnp.float32) + b1_ref[...]
    u = jnp.maximum(u, 0.0)
    t = jnp.dot(u, w2_ref[...], preferred_element_type=jnp.float32) + b2_ref[...]
    if relu_out:
      t = jnp.maximum(t, 0.0)
    if twin:
      wt_ref, bt_ref, o_ref, o2_ref = rest
      tp = jnp.dot(u, wt_ref[...], preferred_element_type=jnp.float32) + bt_ref[...]
      if relu_out:
        tp = jnp.maximum(tp, 0.0)
      o2_ref[...] = tp.astype(jnp.bfloat16)
    else:
      (o_ref,) = rest
    o_ref[...] = t

  in_specs = [
      pl.BlockSpec((1, 1), lambda i: (0, 0)),
      pl.BlockSpec((_RB, EMB), lambda i: (i, 0)),
      pl.BlockSpec((NC, _RB, EMB), lambda i: (0, i, 0)),
      pl.BlockSpec((EMB, 2 * EMB), lambda i: (0, 0)),
      pl.BlockSpec((1, 2 * EMB), lambda i: (0, 0)),
      pl.BlockSpec((2 * EMB, EMB), lambda i: (0, 0)),
      pl.BlockSpec((1, EMB), lambda i: (0, 0)),
  ]
  args = [alpha, h, parts, W1f, b1f[None], W2f, b2f[None]]
  out_specs = pl.BlockSpec((_RB, EMB), lambda i: (i, 0))
  out_shape = jax.ShapeDtypeStruct((N_NODES, EMB), jnp.float32)
  if twin:
    in_specs += [pl.BlockSpec((2 * EMB, EMB), lambda i: (0, 0)),
                 pl.BlockSpec((1, EMB), lambda i: (0, 0))]
    args += [Wtw, btw[None]]
    out_specs = [out_specs, pl.BlockSpec((_RB, EMB), lambda i: (i, 0))]
    out_shape = [out_shape,
                 jax.ShapeDtypeStruct((N_NODES, EMB), jnp.bfloat16)]

  return pl.pallas_call(
      body,
      grid=(N_NODES // _RB,),
      in_specs=in_specs,
      out_specs=out_specs,
      out_shape=out_shape,
  )(*args)


def kernel(x, edge_index, edge_attr, We, be, eps, W1, b1, W2, b2,
           g1, bb1, m1, v1, go, bo, mo, vo):
  s1 = g1 / jnp.sqrt(v1 + 1e-5)
  W1f = W1 * s1[:, None, :]
  b1f = (b1 - m1) * s1 + bb1
  so = go / jnp.sqrt(vo + 1e-5)
  W2f = W2 * so[:, None, :]
  b2f = (b2 - mo) * so + bo

  Wcat = jnp.concatenate([We[0], We[1]], axis=1)
  bcat = jnp.concatenate([be[0], be[1]])[None, :]
  E1, E2 = _edge_embed(edge_attr, Wcat, bcat)

  src = edge_index[0]
  dst = edge_index[1]
  z = jnp.zeros((RPT, EMB), jnp.float32)

  parts = _sc_segment(x, E1, src, dst, z).reshape(NC, NPAD, EMB)
  h = _mlp(x, parts, (1.0 + eps[0]).reshape(1, 1),
           W1f[0], b1f[0], W2f[0], b2f[0], relu_out=True)
  parts = _sc_segment(h, E2, src, dst, z).reshape(NC, NPAD, EMB)
  h = _mlp(h, parts, (1.0 + eps[1]).reshape(1, 1),
           W1f[1], b1f[1], W2f[1], b2f[1], relu_out=False)
  return h


# speedup vs baseline: 2.5014x; 1.2265x over previous
import functools

import jax
import jax.numpy as jnp
import numpy as np
from jax import lax
from jax.experimental import pallas as pl
from jax.experimental.pallas import tpu as pltpu
from jax.experimental.pallas import tpu_sc as plsc

N_NODES = 10000
N_EDGES = 320000
D_EDGE = 16
EMB = 128

NC = 2
NS = 16
NW = NC * NS
EPW = N_EDGES // NW
CH = 80
NCHUNK = EPW // CH
NPAD = 10240
RPT = NPAD // NS

_EB = 4000
_RB = 1000

def _edge_embed(edge_attr, Wcat, bcat):
  def body(a_ref, w_ref, b_ref, o1_ref, o2_ref):
    e = jnp.dot(a_ref[...], w_ref[...],
                preferred_element_type=jnp.float32) + b_ref[...]
    o1_ref[...] = e[:, :EMB].astype(jnp.bfloat16)
    o2_ref[...] = e[:, EMB:].astype(jnp.bfloat16)

  return pl.pallas_call(
      body,
      grid=(N_EDGES // _EB,),
      in_specs=[
          pl.BlockSpec((_EB, D_EDGE), lambda i: (i, 0)),
          pl.BlockSpec((D_EDGE, 2 * EMB), lambda i: (0, 0)),
          pl.BlockSpec((1, 2 * EMB), lambda i: (0, 0)),
      ],
      out_specs=[
          pl.BlockSpec((_EB, EMB), lambda i: (i, 0)),
          pl.BlockSpec((_EB, EMB), lambda i: (i, 0)),
      ],
      out_shape=[jax.ShapeDtypeStruct((N_EDGES, EMB), jnp.bfloat16)] * 2,
  )(edge_attr, Wcat, bcat)


_SC_MESH = plsc.VectorSubcoreMesh(core_axis_name="c", subcore_axis_name="s")


@functools.partial(
    pl.kernel,
    out_type=jax.ShapeDtypeStruct((NC * NPAD, EMB), jnp.float32),
    mesh=_SC_MESH,
    scratch_types=[
        pltpu.VMEM((2, CH), jnp.int32),
        pltpu.VMEM((2, CH), jnp.int32),
        pltpu.VMEM((2, CH), jnp.int32),
        pltpu.VMEM((3, CH, EMB), jnp.float32),
        pltpu.VMEM((2, CH // 2, 2, EMB), jnp.bfloat16),
        pltpu.VMEM_SHARED((NPAD, EMB), jnp.float32),
        [pltpu.SemaphoreType.DMA] * 3,
        [pltpu.SemaphoreType.DMA] * 2,
        [pltpu.SemaphoreType.DMA] * 2,
        [pltpu.SemaphoreType.DMA] * 2,
        [pltpu.SemaphoreType.DMA] * 1,
    ],
)
def _sc_segment(h_hbm, e_hbm, src_hbm, dst_hbm, z_hbm, out_hbm,
                sidx, didx, sdst, rows, ebuf, aggsh,
                gsem, esem, isem, ssem, zsem):
  c = lax.axis_index("c")
  s = lax.axis_index("s")
  wid = c * NS + s
  ebase = wid * EPW
  ebase2 = wid * (EPW // 2)
  CH2 = CH // 2

  zcp = pltpu.async_copy(z_hbm, aggsh.at[pl.ds(s * RPT, RPT)], zsem[0])

  pltpu.sync_copy(src_hbm.at[pl.ds(ebase, CH)], sidx.at[0])
  pltpu.sync_copy(dst_hbm.at[pl.ds(ebase, CH)], didx.at[0])
  pltpu.async_copy(src_hbm.at[pl.ds(ebase + CH, CH)], sidx.at[1], isem[1])
  pltpu.async_copy(dst_hbm.at[pl.ds(ebase + CH, CH)], didx.at[1], isem[1])
  for k in range(2):
    pltpu.async_copy(e_hbm.at[pl.ds(ebase2 + k * CH2, CH2)], ebuf.at[k],
                     esem[k])
  pltpu.async_copy(h_hbm.at[sidx.at[0]], rows.at[0], gsem[0])
  zcp.wait()
  plsc.subcore_barrier()

  def _do_chunk(t, r3, r2, first, e_ok, g_ok, i_ok):
    r3p = (r3 + 2) % 3
    r3n = (r3 + 1) % 3
    r2n = 1 - r2

    pltpu.make_async_copy(h_hbm.at[pl.ds(0, CH)], rows.at[r3], gsem[r3]).wait()

    if g_ok:
      pltpu.make_async_copy(src_hbm.at[pl.ds(0, CH)], sidx.at[r2n],
                            isem[r2n]).wait()
      pltpu.make_async_copy(dst_hbm.at[pl.ds(0, CH)], didx.at[r2n],
                            isem[r2n]).wait()
      pltpu.async_copy(h_hbm.at[sidx.at[r2n]], rows.at[r3n], gsem[r3n])

    if not first:
      pltpu.make_async_copy(out_hbm.at[pl.ds(0, CH)], rows.at[r3p],
                            ssem[r2n]).wait()

    pltpu.make_async_copy(e_hbm.at[pl.ds(0, CH2)], ebuf.at[r2],
                          esem[r2]).wait()

    @pl.loop(0, CH2, unroll=4)
    def _row(r):
      x = ebuf[r2, r].astype(jnp.float32)
      rows[r3, 2 * r] = jnp.maximum(rows[r3, 2 * r] + x[0], 0.0)
      rows[r3, 2 * r + 1] = jnp.maximum(rows[r3, 2 * r + 1] + x[1], 0.0)

    if e_ok:
      pltpu.async_copy(e_hbm.at[pl.ds(ebase2 + (t + 2) * CH2, CH2)],
                       ebuf.at[r2], esem[r2])

    for k in range(CH // 16):
      sl = pl.ds(k * 16, 16)
      sdst[r2, sl] = didx[r2, sl]
    pltpu.async_copy(rows.at[r3], aggsh.at[sdst.at[r2]], ssem[r2], add=True)

    if i_ok:
      nbase = ebase + (t + 2) * CH
      pltpu.async_copy(src_hbm.at[pl.ds(nbase, CH)], sidx.at[r2], isem[r2])
      pltpu.async_copy(dst_hbm.at[pl.ds(nbase, CH)], didx.at[r2], isem[r2])

  for t in range(6):
    _do_chunk(t, t % 3, t % 2, t == 0, True, True, True)

  @pl.loop(1, NCHUNK // 6)
  def _grp(i):
    t0 = 6 * i
    for j in range(6):
      _do_chunk(t0 + j, j % 3, j % 2, False, True, True, True)

  for t in range(6 * (NCHUNK // 6), NCHUNK):
    _do_chunk(t, t % 3, t % 2, False, t + 2 < NCHUNK, t + 1 < NCHUNK,
              t + 2 < NCHUNK)

  pltpu.make_async_copy(out_hbm.at[pl.ds(0, CH)],
                        rows.at[(NCHUNK - 1) % 3],
                        ssem[(NCHUNK - 1) % 2]).wait()

  plsc.subcore_barrier()
  pltpu.sync_copy(aggsh.at[pl.ds(s * RPT, RPT)],
                  out_hbm.at[pl.ds(c * NPAD + s * RPT, RPT)])


def _mlp(h, parts, alpha, W1f, b1f, W2f, b2f, relu_out, Wtw=None, btw=None):
  twin = Wtw is not None

  def body(al_ref, h_ref, p_ref, w1_ref, b1_ref, w2_ref, b2_ref,
           *rest):
    t = h_ref[...] * al_ref[0, 0] + p_ref[0] + p_ref[1]
    u = jnp.dot(t, w1_ref[...], preferred_element_type=jnp.float32) + b1_ref[...]
    u = jnp.maximum(u, 0.0)
    t = jnp.dot(u, w2_ref[...], preferred_element_type=jnp.float32) + b2_ref[...]
    if relu_out:
      t = jnp.maximum(t, 0.0)
    if twin:
      wt_ref, bt_ref, o_ref, o2_ref = rest
      tp = jnp.dot(u, wt_ref[...], preferred_element_type=jnp.float32) + bt_ref[...]
      if relu_out:
        tp = jnp.maximum(tp, 0.0)
      o2_ref[...] = tp.astype(jnp.bfloat16)
    else:
      (o_ref,) = rest
    o_ref[...] = t

  in_specs = [
      pl.BlockSpec((1, 1), lambda i: (0, 0)),
      pl.BlockSpec((_RB, EMB), lambda i: (i, 0)),
      pl.BlockSpec((NC, _RB, EMB), lambda i: (0, i, 0)),
      pl.BlockSpec((EMB, 2 * EMB), lambda i: (0, 0)),
      pl.BlockSpec((1, 2 * EMB), lambda i: (0, 0)),
      pl.BlockSpec((2 * EMB, EMB), lambda i: (0, 0)),
      pl.BlockSpec((1, EMB), lambda i: (0, 0)),
  ]
  args = [alpha, h, parts, W1f, b1f[None], W2f, b2f[None]]
  out_specs = pl.BlockSpec((_RB, EMB), lambda i: (i, 0))
  out_shape = jax.ShapeDtypeStruct((N_NODES, EMB), jnp.float32)
  if twin:
    in_specs += [pl.BlockSpec((2 * EMB, EMB), lambda i: (0, 0)),
                 pl.BlockSpec((1, EMB), lambda i: (0, 0))]
    args += [Wtw, btw[None]]
    out_specs = [out_specs, pl.BlockSpec((_RB, EMB), lambda i: (i, 0))]
    out_shape = [out_shape,
                 jax.ShapeDtypeStruct((N_NODES, EMB), jnp.bfloat16)]

  return pl.pallas_call(
      body,
      grid=(N_NODES // _RB,),
      in_specs=in_specs,
      out_specs=out_specs,
      out_shape=out_shape,
  )(*args)


def kernel(x, edge_index, edge_attr, We, be, eps, W1, b1, W2, b2,
           g1, bb1, m1, v1, go, bo, mo, vo):
  s1 = g1 / jnp.sqrt(v1 + 1e-5)
  W1f = W1 * s1[:, None, :]
  b1f = (b1 - m1) * s1 + bb1
  so = go / jnp.sqrt(vo + 1e-5)
  W2f = W2 * so[:, None, :]
  b2f = (b2 - mo) * so + bo

  Wcat = jnp.concatenate([We[0], We[1]], axis=1)
  bcat = jnp.concatenate([be[0], be[1]])[None, :]
  E1, E2 = _edge_embed(edge_attr, Wcat, bcat)
  E1 = E1.reshape(N_EDGES // 2, 2, EMB)
  E2 = E2.reshape(N_EDGES // 2, 2, EMB)

  src = edge_index[0]
  dst = edge_index[1]
  z = jnp.zeros((RPT, EMB), jnp.float32)

  parts = _sc_segment(x, E1, src, dst, z).reshape(NC, NPAD, EMB)
  h = _mlp(x, parts, (1.0 + eps[0]).reshape(1, 1),
           W1f[0], b1f[0], W2f[0], b2f[0], relu_out=True)
  parts = _sc_segment(h, E2, src, dst, z).reshape(NC, NPAD, EMB)
  h = _mlp(h, parts, (1.0 + eps[1]).reshape(1, 1),
           W1f[1], b1f[1], W2f[1], b2f[1], relu_out=False)
  return h
